# baseline (speedup 1.0000x reference)
import functools

import jax
import jax.numpy as jnp
import numpy as np
from jax import lax
from jax.experimental import pallas as pl
from jax.experimental.pallas import tpu as pltpu
from jax.experimental.pallas import tpu_sc as plsc

N = 10000
E = 160000
E2 = 320000

BE = 640
BN = 1000

_F32 = jnp.float32

_DIV = np.exp(np.log(10000.0) * (2.0 * np.arange(8, dtype=np.float32) / 16.0))
_ANG_SCALE = (8.0 / _DIV).astype(np.float32)
_PE_PERM = np.concatenate([np.arange(0, 16, 2), np.arange(1, 16, 2)])


def _silu(v):
    return v * jax.nn.sigmoid(v)


def _row_spec(b, w):
    return pl.BlockSpec((b, w), lambda i: (i, 0))


def _full_spec(shape):
    nd = len(shape)
    return pl.BlockSpec(shape, lambda i: (0,) * nd)


def _efn_body(hr, hc, w1r, w1c, b1, w2, b2, out):
    t = (jnp.dot(hr[...], w1r[...], preferred_element_type=_F32)
         + jnp.dot(hc[...], w1c[...], preferred_element_type=_F32) + b1[...])
    t = _silu(t)
    out[...] = jnp.dot(t, w2[...], preferred_element_type=_F32) + b2[...]


def _tc_efn(hr, hc, w1r, w1c, b1, w2, b2):
    grid = (E // BE,)
    return pl.pallas_call(
        _efn_body,
        grid=grid,
        in_specs=[_row_spec(BE, 128), _row_spec(BE, 128),
                  _full_spec((128, 128)), _full_spec((128, 128)),
                  _full_spec((1, 128)), _full_spec((128, 128)),
                  _full_spec((1, 128))],
        out_specs=_row_spec(BE, 128),
        out_shape=jax.ShapeDtypeStruct((E, 128), _F32),
    )(hr, hc, w1r, w1c, b1, w2, b2)


def _m1m2_body(prod, ff, angs, p1pe, p1ff, p1b, p2pe, p2ff, p2b,
               sincos_out, m1_out, m2_out):
    vtv = jnp.sum(prod[...], axis=1, keepdims=True)
    ang = vtv * angs[...]
    sincos = jnp.concatenate([jnp.sin(ang), jnp.cos(ang)], axis=1)
    sincos_out[...] = sincos
    ffv = ff[...]
    m1 = (jnp.dot(sincos, p1pe[...], preferred_element_type=_F32)
          + jnp.dot(ffv, p1ff[...], preferred_element_type=_F32) + p1b[...])
    m1_out[...] = _silu(m1)
    m2 = (jnp.dot(sincos, p2pe[...], preferred_element_type=_F32)
          + jnp.dot(ffv, p2ff[...], preferred_element_type=_F32) + p2b[...])
    m2_out[...] = _silu(m2)


def _tc_m1m2(prod, ff, angs, p1pe, p1ff, p1b, p2pe, p2ff, p2b):
    grid = (E2 // BE,)
    return pl.pallas_call(
        _m1m2_body,
        grid=grid,
        in_specs=[_row_spec(BE, 16), _row_spec(BE, 128), _full_spec((1, 8)),
                  _full_spec((16, 128)), _full_spec((128, 128)),
                  _full_spec((1, 128)),
                  _full_spec((16, 128)), _full_spec((128, 128)),
                  _full_spec((1, 128))],
        out_specs=[_row_spec(BE, 16), _row_spec(BE, 128), _row_spec(BE, 128)],
        out_shape=[jax.ShapeDtypeStruct((E2, 16), _F32),
                   jax.ShapeDtypeStruct((E2, 128), _F32),
                   jax.ShapeDtypeStruct((E2, 128), _F32)],
    )(prod, ff, angs, p1pe, p1ff, p1b, p2pe, p2ff, p2b)


def _nbx2_body(sincos, ff, mm, p3pe, p3ff, p3mm, p3b, out):
    out[...] = (jnp.dot(sincos[...], p3pe[...], preferred_element_type=_F32)
                + jnp.dot(ff[...], p3ff[...], preferred_element_type=_F32)
                + jnp.dot(mm[...], p3mm[...], preferred_element_type=_F32)
                + p3b[...])


def _tc_nbx2(sincos, ff, mm, p3pe, p3ff, p3mm, p3b):
    grid = (E2 // BE,)
    return pl.pallas_call(
        _nbx2_body,
        grid=grid,
        in_specs=[_row_spec(BE, 16), _row_spec(BE, 128), _row_spec(BE, 128),
                  _full_spec((16, 128)), _full_spec((128, 128)),
                  _full_spec((128, 128)), _full_spec((1, 128))],
        out_specs=_row_spec(BE, 128),
        out_shape=jax.ShapeDtypeStruct((E2, 128), _F32),
    )(sincos, ff, mm, p3pe, p3ff, p3mm, p3b)


def _t1_body(sr, sc, cd, ign_r, ign_c, ignb, cw1, cb1, cw2, one3,
             t1_out, trans_out):
    t1 = (jnp.dot(sr[...], ign_r[...], preferred_element_type=_F32)
          + jnp.dot(sc[...], ign_c[...], preferred_element_type=_F32)
          + ignb[...])
    t1_out[...] = t1
    u = _silu(jnp.dot(t1, cw1[...], preferred_element_type=_F32) + cb1[...])
    w = jnp.dot(u, cw2[...], preferred_element_type=_F32)
    trans_out[...] = cd[...] * w + one3[...]


def _tc_t1(sr, sc, cd, ign_r, ign_c, ignb, cw1, cb1, cw2, one3):
    grid = (E // BE,)
    return pl.pallas_call(
        _t1_body,
        grid=grid,
        in_specs=[_row_spec(BE, 128), _row_spec(BE, 128), _row_spec(BE, 16),
                  _full_spec((128, 128)), _full_spec((128, 128)),
                  _full_spec((1, 128)), _full_spec((128, 128)),
                  _full_spec((1, 128)), _full_spec((128, 1)),
                  _full_spec((1, 16))],
        out_specs=[_row_spec(BE, 128), _row_spec(BE, 16)],
        out_shape=[jax.ShapeDtypeStruct((E, 128), _F32),
                   jax.ShapeDtypeStruct((E, 16), _F32)],
    )(sr, sc, cd, ign_r, ign_c, ignb, cw1, cb1, cw2, one3)


def _final_body(x16, aggc, t0, h, nw1, nb1, nw2, nb2, h_out, x_out):
    cnt = jnp.maximum(aggc[...][:, 3:4], 1.0)
    x_out[...] = x16[...] + aggc[...] * (1.0 / cnt)
    u = _silu(jnp.dot(t0[...], nw1[...], preferred_element_type=_F32)
              + nb1[...])
    h_out[...] = h[...] + jnp.dot(u, nw2[...],
                                  preferred_element_type=_F32) + nb2[...]


def _tc_final(x16, aggc, t0, h, nw1, nb1, nw2, nb2):
    grid = (N // BN,)
    return pl.pallas_call(
        _final_body,
        grid=grid,
        in_specs=[_row_spec(BN, 16), _row_spec(BN, 16), _row_spec(BN, 128),
                  _row_spec(BN, 128),
                  _full_spec((128, 128)), _full_spec((1, 128)),
                  _full_spec((128, 128)), _full_spec((1, 128))],
        out_specs=[_row_spec(BN, 128), _row_spec(BN, 16)],
        out_shape=[jax.ShapeDtypeStruct((N, 128), _F32),
                   jax.ShapeDtypeStruct((N, 16), _F32)],
    )(x16, aggc, t0, h, nw1, nb1, nw2, nb2)


_NC, _NS = 2, 16
_NW = _NC * _NS


def _sc_gather2(t1, t2, idx1, idx2, op, cb):
    B = idx1.shape[0]
    D = t1.shape[1]
    cnt = B // _NW
    nch = cnt // cb
    assert cnt % cb == 0 and cb % 8 == 0 and cb <= 128, (B, cnt, cb)
    mesh = plsc.VectorSubcoreMesh(core_axis_name="c", subcore_axis_name="s")
    nout = 2 if op == 'none' else 1
    if nout == 2:
        outs = (jax.ShapeDtypeStruct((B, D), _F32),
                jax.ShapeDtypeStruct((B, D), _F32))
    else:
        outs = jax.ShapeDtypeStruct((B, D), _F32)

    def body(t1_h, t2_h, i1_h, i2_h, *rest):
        if nout == 2:
            o1_h, o2_h = rest[:2]
            iv1, iv2, b1, b2, sm1, sm2 = rest[2:]
        else:
            o1_h = rest[0]
            o2_h = None
            iv1, iv2, b1, b2, sm1, sm2 = rest[1:]
        wid = lax.axis_index("s") * _NC + lax.axis_index("c")
        base0 = wid * cnt

        def step(k, carry):
            base = base0 + k * cb
            pltpu.sync_copy(i1_h.at[pl.ds(base, cb)], iv1)
            pltpu.sync_copy(i2_h.at[pl.ds(base, cb)], iv2)
            c1 = pltpu.async_copy(t1_h.at[iv1], b1, sm1)
            c2 = pltpu.async_copy(t2_h.at[iv2], b2, sm2)
            c1.wait()
            c2.wait()
            if op != 'none':
                def ew(i, c2_):
                    for g in range(D // 16):
                        sl = (i, pl.ds(g * 16, 16))
                        if op == 'mul':
                            b1[sl] = b1[sl] * b2[sl]
                        else:
                            b1[sl] = b1[sl] - b2[sl]
                    return c2_
                lax.fori_loop(0, cb, ew, 0)
            pltpu.sync_copy(b1, o1_h.at[pl.ds(base, cb)])
            if nout == 2:
                pltpu.sync_copy(b2, o2_h.at[pl.ds(base, cb)])
            return carry

        lax.fori_loop(0, nch, step, 0)

    f = pl.kernel(
        body, out_type=outs, mesh=mesh,
        compiler_params=pltpu.CompilerParams(use_tc_tiling_on_sc=False),
        scratch_types=[
            pltpu.VMEM((cb,), jnp.int32), pltpu.VMEM((cb,), jnp.int32),
            pltpu.VMEM((cb, D), _F32), pltpu.VMEM((cb, D), _F32),
            pltpu.SemaphoreType.DMA, pltpu.SemaphoreType.DMA])
    return f(t1, t2, idx1, idx2)


def _sc_scatter_pair(src1, idx1, src2, idx2, T):
    B = idx1.shape[0]
    per = B // _NS
    cbs = 80
    nch = per // cbs
    zr = T // _NS
    zc = 1000
    nzc = zr // zc
    assert per % cbs == 0 and zr % zc == 0
    mesh = plsc.VectorSubcoreMesh(core_axis_name="c", subcore_axis_name="s")
    outs = (jax.ShapeDtypeStruct((T, 128), _F32),
            jax.ShapeDtypeStruct((T, 128), _F32))

    def body(s1_h, i1_h, s2_h, i2_h, z_h, o1_h, o2_h, table, iv, sv, zb):
        cid = lax.axis_index("c")
        sid = lax.axis_index("s")
        pltpu.sync_copy(z_h, zb)
        for srci in range(2):
            s_h = (s1_h, s2_h)[srci]
            i_h = (i1_h, i2_h)[srci]
            o_h = (o1_h, o2_h)[srci]

            @pl.when(cid == srci)
            def _():
                for cc in range(16):
                    col = cc * 8

                    def zstep(j, c):
                        pltpu.sync_copy(
                            zb, table.at[pl.ds(sid * zr + j * zc, zc)])
                        return c
                    lax.fori_loop(0, nzc, zstep, 0)
                    plsc.subcore_barrier()

                    def sstep(k, c):
                        base = sid * per + k * cbs
                        pltpu.sync_copy(i_h.at[pl.ds(base, cbs)], iv)
                        pltpu.sync_copy(
                            s_h.at[pl.ds(base, cbs), pl.ds(col, 8)], sv)
                        pltpu.sync_copy(sv, table.at[iv], add=True)
                        return c
                    lax.fori_loop(0, nch, sstep, 0)
                    plsc.subcore_barrier()
                    pltpu.sync_copy(
                        table.at[pl.ds(sid * zr, zr)],
                        o_h.at[pl.ds(sid * zr, zr), pl.ds(col, 8)])
                    plsc.subcore_barrier()

    f = pl.kernel(
        body, out_type=outs, mesh=mesh,
        compiler_params=pltpu.CompilerParams(use_tc_tiling_on_sc=False),
        scratch_types=[
            pltpu.VMEM_SHARED((T, 8), _F32),
            pltpu.VMEM((cbs,), jnp.int32),
            pltpu.VMEM((cbs, 8), _F32),
            pltpu.VMEM((zc, 8), _F32)])
    z = jnp.zeros((zc, 8), _F32)
    return f(src1, idx1, src2, idx2, z)


def _sc_scatter_node(vals128, vals16, idx, T):
    B = idx.shape[0]
    per = B // _NS
    cbs = 80
    nch = per // cbs
    zr = T // _NS
    assert per % cbs == 0 and T % _NS == 0
    mesh = plsc.VectorSubcoreMesh(core_axis_name="c", subcore_axis_name="s")
    outs = (jax.ShapeDtypeStruct((T, 128), _F32),
            jax.ShapeDtypeStruct((T, 16), _F32))

    def body(v128_h, v16_h, i_h, z_h, o128_h, o16_h,
             tbl128, tbl16, iv, sv128, sv16):
        cid = lax.axis_index("c")
        sid = lax.axis_index("s")

        @pl.when(cid == 0)
        def _():
            for j in range(zr // 125):
                pltpu.sync_copy(
                    z_h, tbl128.at[pl.ds(sid * zr + j * 125, 125)])
            plsc.subcore_barrier()

            def sstep(k, c):
                base = sid * per + k * cbs
                pltpu.sync_copy(i_h.at[pl.ds(base, cbs)], iv)
                pltpu.sync_copy(v128_h.at[pl.ds(base, cbs)], sv128)
                pltpu.sync_copy(sv128, tbl128.at[iv], add=True)
                return c
            lax.fori_loop(0, nch, sstep, 0)
            plsc.subcore_barrier()
            pltpu.sync_copy(tbl128.at[pl.ds(sid * zr, zr)],
                            o128_h.at[pl.ds(sid * zr, zr)])

        @pl.when(cid == 1)
        def _():
            for j in range(zr // 125):
                pltpu.sync_copy(
                    z_h.at[:, pl.ds(0, 16)],
                    tbl16.at[pl.ds(sid * zr + j * 125, 125)])
            plsc.subcore_barrier()

            def sstep(k, c):
                base = sid * per + k * cbs
                pltpu.sync_copy(i_h.at[pl.ds(base, cbs)], iv)
                pltpu.sync_copy(v16_h.at[pl.ds(base, cbs)], sv16)
                pltpu.sync_copy(sv16, tbl16.at[iv], add=True)
                return c
            lax.fori_loop(0, nch, sstep, 0)
            plsc.subcore_barrier()
            pltpu.sync_copy(tbl16.at[pl.ds(sid * zr, zr)],
                            o16_h.at[pl.ds(sid * zr, zr)])

    f = pl.kernel(
        body, out_type=outs, mesh=mesh,
        compiler_params=pltpu.CompilerParams(use_tc_tiling_on_sc=False),
        scratch_types=[
            pltpu.VMEM_SHARED((T, 128), _F32),
            pltpu.VMEM_SHARED((T, 16), _F32),
            pltpu.VMEM((cbs,), jnp.int32),
            pltpu.VMEM((cbs, 128), _F32),
            pltpu.VMEM((cbs, 16), _F32)])
    z = jnp.zeros((125, 128), _F32)
    return f(vals128, vals16, idx, z)


def _gather_rows(table, idx):
    return jnp.take(table, idx, axis=0)


def _segsum(vals, idx, num):
    return jax.ops.segment_sum(vals, idx, num_segments=num)


def kernel(h, x, edges, nb_edge, edge_attr, nb_num_nodes, params):
    del nb_num_nodes
    rows, cols = edges[0], edges[1]
    nbr, nbc = nb_edge[0], nb_edge[1]

    x16 = jnp.pad(x, ((0, 0), (0, 13)))

    w1r = params['ee_W1'][:128]
    w1c = params['ee_W1'][128:]
    b1 = params['ee_b1'].reshape(1, 128)
    w2 = params['ee_W2']
    b2 = params['ee_b2'].reshape(1, 128)
    p1pe = params['p1_W'][:16][_PE_PERM]
    p1ff = params['p1_W'][16:]
    p1b = params['p1_b'].reshape(1, 128)
    p2pe = params['p2_W'][:16][_PE_PERM]
    p2ff = params['p2_W'][16:]
    p2b = params['p2_b'].reshape(1, 128)
    p3pe = params['p3_W'][:16][_PE_PERM]
    p3ff = params['p3_W'][16:144]
    p3mm = params['p3_W'][144:]
    p3b = params['p3_b'].reshape(1, 128)
    ign_r = params['ign_W'][:128]
    ign_c = params['ign_W'][128:]
    ignb = params['ign_b'].reshape(1, 128)
    cw1 = params['cm_W1']
    cb1 = params['cm_b1'].reshape(1, 128)
    cw2 = params['cm_W2']
    nw1 = params['nd_W1']
    nb1 = params['nd_b1'].reshape(1, 128)
    nw2 = params['nd_W2']
    nb2 = params['nd_b2'].reshape(1, 128)
    one3 = jnp.zeros((1, 16), _F32).at[0, 3].set(1.0)
    angs = jnp.asarray(_ANG_SCALE).reshape(1, 8)

    cd = _sc_gather2(x16, x16, rows, cols, 'sub', 40)
    prod = _sc_gather2(cd, cd, nbr, nbc, 'mul', 80)
    hr, hc = _sc_gather2(h, h, rows, cols, 'none', 40)
    efn = _tc_efn(hr, hc, w1r, w1c, b1, w2, b2)
    ff = _sc_gather2(efn, efn, nbr, nbc, 'mul', 80)
    sincos, m1, m2 = _tc_m1m2(prod, ff, angs, p1pe, p1ff, p1b,
                              p2pe, p2ff, p2b)
    s1, s2 = _sc_scatter_pair(m1, nbr, m2, nbc, E)
    mm = _sc_gather2(s1, s2, nbr, nbc, 'mul', 80)
    nb_x2 = _tc_nbx2(sincos, ff, mm, p3pe, p3ff, p3mm, p3b)
    sr, sc = _sc_scatter_pair(nb_x2, nbr, nb_x2, nbc, E)
    t1, trans16 = _tc_t1(sr, sc, cd, ign_r, ign_c, ignb, cw1, cb1, cw2, one3)
    t0, aggc = _sc_scatter_node(t1, trans16, rows, N)
    h_out, x16_out = _tc_final(x16, aggc, t0, h, nw1, nb1, nw2, nb2)
    return (h_out, x16_out[:, :3], edge_attr)


# speedup vs baseline: 2.0642x; 2.0642x over previous
import functools

import jax
import jax.numpy as jnp
import numpy as np
from jax import lax
from jax.experimental import pallas as pl
from jax.experimental.pallas import tpu as pltpu
from jax.experimental.pallas import tpu_sc as plsc

N = 10000
E = 160000
E2 = 320000

BE = 640
BN = 1000

_F32 = jnp.float32

_DIV = np.exp(np.log(10000.0) * (2.0 * np.arange(8, dtype=np.float32) / 16.0))
_ANG_SCALE = (8.0 / _DIV).astype(np.float32)
_PE_PERM = np.concatenate([np.arange(0, 16, 2), np.arange(1, 16, 2)])


def _silu(v):
    return v * jax.nn.sigmoid(v)


def _row_spec(b, w):
    return pl.BlockSpec((b, w), lambda i: (i, 0))


def _full_spec(shape):
    nd = len(shape)
    return pl.BlockSpec(shape, lambda i: (0,) * nd)


def _efn_body(hr, hc, w1r, w1c, b1, w2, b2, out):
    t = (jnp.dot(hr[...], w1r[...], preferred_element_type=_F32)
         + jnp.dot(hc[...], w1c[...], preferred_element_type=_F32) + b1[...])
    t = _silu(t)
    out[...] = jnp.dot(t, w2[...], preferred_element_type=_F32) + b2[...]


def _tc_efn(hr, hc, w1r, w1c, b1, w2, b2):
    grid = (E // BE,)
    return pl.pallas_call(
        _efn_body,
        grid=grid,
        in_specs=[_row_spec(BE, 128), _row_spec(BE, 128),
                  _full_spec((128, 128)), _full_spec((128, 128)),
                  _full_spec((1, 128)), _full_spec((128, 128)),
                  _full_spec((1, 128))],
        out_specs=_row_spec(BE, 128),
        out_shape=jax.ShapeDtypeStruct((E, 128), _F32),
    )(hr, hc, w1r, w1c, b1, w2, b2)


def _m1m2_body(prod, ff, angs, p1pe, p1ff, p1b, p2pe, p2ff, p2b,
               sincos_out, m1_out, m2_out):
    vtv = jnp.sum(prod[...], axis=1, keepdims=True)
    ang = vtv * angs[...]
    sincos = jnp.concatenate([jnp.sin(ang), jnp.cos(ang)], axis=1)
    sincos_out[...] = sincos
    ffv = ff[...]
    m1 = (jnp.dot(sincos, p1pe[...], preferred_element_type=_F32)
          + jnp.dot(ffv, p1ff[...], preferred_element_type=_F32) + p1b[...])
    m1_out[...] = _silu(m1)
    m2 = (jnp.dot(sincos, p2pe[...], preferred_element_type=_F32)
          + jnp.dot(ffv, p2ff[...], preferred_element_type=_F32) + p2b[...])
    m2_out[...] = _silu(m2)


def _tc_m1m2(prod, ff, angs, p1pe, p1ff, p1b, p2pe, p2ff, p2b):
    grid = (E2 // BE,)
    return pl.pallas_call(
        _m1m2_body,
        grid=grid,
        in_specs=[_row_spec(BE, 16), _row_spec(BE, 128), _full_spec((1, 8)),
                  _full_spec((16, 128)), _full_spec((128, 128)),
                  _full_spec((1, 128)),
                  _full_spec((16, 128)), _full_spec((128, 128)),
                  _full_spec((1, 128))],
        out_specs=[_row_spec(BE, 16), _row_spec(BE, 128), _row_spec(BE, 128)],
        out_shape=[jax.ShapeDtypeStruct((E2, 16), _F32),
                   jax.ShapeDtypeStruct((E2, 128), _F32),
                   jax.ShapeDtypeStruct((E2, 128), _F32)],
    )(prod, ff, angs, p1pe, p1ff, p1b, p2pe, p2ff, p2b)


def _nbx2_body(sincos, ff, mm, p3pe, p3ff, p3mm, p3b, out):
    out[...] = (jnp.dot(sincos[...], p3pe[...], preferred_element_type=_F32)
                + jnp.dot(ff[...], p3ff[...], preferred_element_type=_F32)
                + jnp.dot(mm[...], p3mm[...], preferred_element_type=_F32)
                + p3b[...])


def _tc_nbx2(sincos, ff, mm, p3pe, p3ff, p3mm, p3b):
    grid = (E2 // BE,)
    return pl.pallas_call(
        _nbx2_body,
        grid=grid,
        in_specs=[_row_spec(BE, 16), _row_spec(BE, 128), _row_spec(BE, 128),
                  _full_spec((16, 128)), _full_spec((128, 128)),
                  _full_spec((128, 128)), _full_spec((1, 128))],
        out_specs=_row_spec(BE, 128),
        out_shape=jax.ShapeDtypeStruct((E2, 128), _F32),
    )(sincos, ff, mm, p3pe, p3ff, p3mm, p3b)


def _t1_body(sr, sc, cd, ign_r, ign_c, ignb, cw1, cb1, cw2, one3,
             t1_out, trans_out):
    t1 = (jnp.dot(sr[...], ign_r[...], preferred_element_type=_F32)
          + jnp.dot(sc[...], ign_c[...], preferred_element_type=_F32)
          + ignb[...])
    t1_out[...] = t1
    u = _silu(jnp.dot(t1, cw1[...], preferred_element_type=_F32) + cb1[...])
    w = jnp.dot(u, cw2[...], preferred_element_type=_F32)
    trans_out[...] = cd[...] * w + one3[...]


def _tc_t1(sr, sc, cd, ign_r, ign_c, ignb, cw1, cb1, cw2, one3):
    grid = (E // BE,)
    return pl.pallas_call(
        _t1_body,
        grid=grid,
        in_specs=[_row_spec(BE, 128), _row_spec(BE, 128), _row_spec(BE, 16),
                  _full_spec((128, 128)), _full_spec((128, 128)),
                  _full_spec((1, 128)), _full_spec((128, 128)),
                  _full_spec((1, 128)), _full_spec((128, 1)),
                  _full_spec((1, 16))],
        out_specs=[_row_spec(BE, 128), _row_spec(BE, 16)],
        out_shape=[jax.ShapeDtypeStruct((E, 128), _F32),
                   jax.ShapeDtypeStruct((E, 16), _F32)],
    )(sr, sc, cd, ign_r, ign_c, ignb, cw1, cb1, cw2, one3)


def _final_body(x16, aggc, t0, h, nw1, nb1, nw2, nb2, h_out, x_out):
    cnt = jnp.maximum(aggc[...][:, 3:4], 1.0)
    x_out[...] = x16[...] + aggc[...] * (1.0 / cnt)
    u = _silu(jnp.dot(t0[...], nw1[...], preferred_element_type=_F32)
              + nb1[...])
    h_out[...] = h[...] + jnp.dot(u, nw2[...],
                                  preferred_element_type=_F32) + nb2[...]


def _tc_final(x16, aggc, t0, h, nw1, nb1, nw2, nb2):
    grid = (N // BN,)
    return pl.pallas_call(
        _final_body,
        grid=grid,
        in_specs=[_row_spec(BN, 16), _row_spec(BN, 16), _row_spec(BN, 128),
                  _row_spec(BN, 128),
                  _full_spec((128, 128)), _full_spec((1, 128)),
                  _full_spec((128, 128)), _full_spec((1, 128))],
        out_specs=[_row_spec(BN, 128), _row_spec(BN, 16)],
        out_shape=[jax.ShapeDtypeStruct((N, 128), _F32),
                   jax.ShapeDtypeStruct((N, 16), _F32)],
    )(x16, aggc, t0, h, nw1, nb1, nw2, nb2)


_NC, _NS = 2, 16
_NW = _NC * _NS


def _sc_gather2(t1, t2, idx1, idx2, op, cb):
    B = idx1.shape[0]
    D = t1.shape[1]
    cnt = B // _NW
    nch = cnt // cb
    assert cnt % cb == 0 and cb % 8 == 0 and cb <= 128, (B, cnt, cb)
    mesh = plsc.VectorSubcoreMesh(core_axis_name="c", subcore_axis_name="s")
    nout = 2 if op == 'none' else 1
    if nout == 2:
        outs = (jax.ShapeDtypeStruct((B, D), _F32),
                jax.ShapeDtypeStruct((B, D), _F32))
    else:
        outs = jax.ShapeDtypeStruct((B, D), _F32)

    def body(t1_h, t2_h, i1_h, i2_h, *rest):
        if nout == 2:
            o1_h, o2_h = rest[:2]
            iv1, iv2, b1, b2, sm1, sm2 = rest[2:]
        else:
            o1_h = rest[0]
            o2_h = None
            iv1, iv2, b1, b2, sm1, sm2 = rest[1:]
        wid = lax.axis_index("s") * _NC + lax.axis_index("c")
        base0 = wid * cnt

        def step(k, carry):
            base = base0 + k * cb
            pltpu.sync_copy(i1_h.at[pl.ds(base, cb)], iv1)
            pltpu.sync_copy(i2_h.at[pl.ds(base, cb)], iv2)
            c1 = pltpu.async_copy(t1_h.at[iv1], b1, sm1)
            c2 = pltpu.async_copy(t2_h.at[iv2], b2, sm2)
            c1.wait()
            c2.wait()
            if op != 'none':
                def ew(i, c2_):
                    for g in range(D // 16):
                        sl = (i, pl.ds(g * 16, 16))
                        if op == 'mul':
                            b1[sl] = b1[sl] * b2[sl]
                        else:
                            b1[sl] = b1[sl] - b2[sl]
                    return c2_
                lax.fori_loop(0, cb, ew, 0)
            pltpu.sync_copy(b1, o1_h.at[pl.ds(base, cb)])
            if nout == 2:
                pltpu.sync_copy(b2, o2_h.at[pl.ds(base, cb)])
            return carry

        lax.fori_loop(0, nch, step, 0)

    f = pl.kernel(
        body, out_type=outs, mesh=mesh,
        compiler_params=pltpu.CompilerParams(use_tc_tiling_on_sc=False),
        scratch_types=[
            pltpu.VMEM((cb,), jnp.int32), pltpu.VMEM((cb,), jnp.int32),
            pltpu.VMEM((cb, D), _F32), pltpu.VMEM((cb, D), _F32),
            pltpu.SemaphoreType.DMA, pltpu.SemaphoreType.DMA])
    return f(t1, t2, idx1, idx2)


def _sc_scatter_pair(src1, idx1, src2, idx2, T):
    B = idx1.shape[0]
    IB = 125
    KB = 20
    NB = B // IB
    nbs = NB // _NS
    nout_ch = nbs // KB
    assert B % IB == 0 and NB % _NS == 0 and nbs % KB == 0
    zr = T // _NS
    zc = 1000
    nzc = zr // zc
    assert zr % zc == 0
    mesh = plsc.VectorSubcoreMesh(core_axis_name="c", subcore_axis_name="s")
    outs = (jax.ShapeDtypeStruct((T, 128), _F32),
            jax.ShapeDtypeStruct((T, 128), _F32))
    i1_2d = idx1.reshape(NB, IB)
    i2_2d = idx2.reshape(NB, IB)

    def body(s1_h, i1_h, s2_h, i2_h, z_h, o1_h, o2_h,
             table, ivb, svb, zb, sem):
        cid = lax.axis_index("c")
        sid = lax.axis_index("s")
        pltpu.sync_copy(z_h, zb)
        for srci in range(2):
            s_h = (s1_h, s2_h)[srci]
            i_h = (i1_h, i2_h)[srci]
            o_h = (o1_h, o2_h)[srci]

            @pl.when(cid == srci)
            def _():
                def cpass(cc, carry):
                    col = cc * 8

                    def zstep(j, c):
                        pltpu.sync_copy(
                            zb, table.at[pl.ds(sid * zr + j * zc, zc)])
                        return c
                    lax.fori_loop(0, nzc, zstep, 0)
                    plsc.subcore_barrier()

                    def sstep(o, c):
                        blk = sid * nbs + o * KB
                        pltpu.sync_copy(i_h.at[pl.ds(blk, KB)], ivb)
                        pltpu.sync_copy(
                            s_h.at[pl.ds(blk * IB, KB * IB), pl.ds(col, 8)],
                            svb)
                        cps = [pltpu.async_copy(
                                   svb.at[pl.ds(b * IB, IB)],
                                   table.at[ivb.at[b]], sem, add=True)
                               for b in range(KB)]
                        for cp in cps:
                            cp.wait()
                        return c
                    lax.fori_loop(0, nout_ch, sstep, 0)
                    plsc.subcore_barrier()
                    pltpu.sync_copy(
                        table.at[pl.ds(sid * zr, zr)],
                        o_h.at[pl.ds(sid * zr, zr), pl.ds(col, 8)])
                    plsc.subcore_barrier()
                    return carry

                lax.fori_loop(0, 16, cpass, 0)

    f = pl.kernel(
        body, out_type=outs, mesh=mesh,
        compiler_params=pltpu.CompilerParams(use_tc_tiling_on_sc=False),
        scratch_types=[
            pltpu.VMEM_SHARED((T, 8), _F32),
            pltpu.VMEM((KB, IB), jnp.int32),
            pltpu.VMEM((KB * IB, 8), _F32),
            pltpu.VMEM((zc, 8), _F32),
            pltpu.SemaphoreType.DMA])
    z = jnp.zeros((zc, 8), _F32)
    return f(src1, i1_2d, src2, i2_2d, z)


def _sc_scatter_node(vals128, vals16, idx, T):
    B = idx.shape[0]
    per = B // _NS
    cbs = 80
    nch = per // cbs
    zr = T // _NS
    assert per % cbs == 0 and T % _NS == 0
    mesh = plsc.VectorSubcoreMesh(core_axis_name="c", subcore_axis_name="s")
    outs = (jax.ShapeDtypeStruct((T, 128), _F32),
            jax.ShapeDtypeStruct((T, 16), _F32))

    def body(v128_h, v16_h, i_h, z_h, o128_h, o16_h,
             tbl128, tbl16, iv, sv128, sv16):
        cid = lax.axis_index("c")
        sid = lax.axis_index("s")

        @pl.when(cid == 0)
        def _():
            for j in range(zr // 125):
                pltpu.sync_copy(
                    z_h, tbl128.at[pl.ds(sid * zr + j * 125, 125)])
            plsc.subcore_barrier()

            def sstep(k, c):
                base = sid * per + k * cbs
                pltpu.sync_copy(i_h.at[pl.ds(base, cbs)], iv)
                pltpu.sync_copy(v128_h.at[pl.ds(base, cbs)], sv128)
                pltpu.sync_copy(sv128, tbl128.at[iv], add=True)
                return c
            lax.fori_loop(0, nch, sstep, 0)
            plsc.subcore_barrier()
            pltpu.sync_copy(tbl128.at[pl.ds(sid * zr, zr)],
                            o128_h.at[pl.ds(sid * zr, zr)])

        @pl.when(cid == 1)
        def _():
            for j in range(zr // 125):
                pltpu.sync_copy(
                    z_h.at[:, pl.ds(0, 16)],
                    tbl16.at[pl.ds(sid * zr + j * 125, 125)])
            plsc.subcore_barrier()

            def sstep(k, c):
                base = sid * per + k * cbs
                pltpu.sync_copy(i_h.at[pl.ds(base, cbs)], iv)
                pltpu.sync_copy(v16_h.at[pl.ds(base, cbs)], sv16)
                pltpu.sync_copy(sv16, tbl16.at[iv], add=True)
                return c
            lax.fori_loop(0, nch, sstep, 0)
            plsc.subcore_barrier()
            pltpu.sync_copy(tbl16.at[pl.ds(sid * zr, zr)],
                            o16_h.at[pl.ds(sid * zr, zr)])

    f = pl.kernel(
        body, out_type=outs, mesh=mesh,
        compiler_params=pltpu.CompilerParams(use_tc_tiling_on_sc=False),
        scratch_types=[
            pltpu.VMEM_SHARED((T, 128), _F32),
            pltpu.VMEM_SHARED((T, 16), _F32),
            pltpu.VMEM((cbs,), jnp.int32),
            pltpu.VMEM((cbs, 128), _F32),
            pltpu.VMEM((cbs, 16), _F32)])
    z = jnp.zeros((125, 128), _F32)
    return f(vals128, vals16, idx, z)


def _gather_rows(table, idx):
    return jnp.take(table, idx, axis=0)


def _segsum(vals, idx, num):
    return jax.ops.segment_sum(vals, idx, num_segments=num)


def kernel(h, x, edges, nb_edge, edge_attr, nb_num_nodes, params):
    del nb_num_nodes
    rows, cols = edges[0], edges[1]
    nbr, nbc = nb_edge[0], nb_edge[1]

    x16 = jnp.pad(x, ((0, 0), (0, 13)))

    w1r = params['ee_W1'][:128]
    w1c = params['ee_W1'][128:]
    b1 = params['ee_b1'].reshape(1, 128)
    w2 = params['ee_W2']
    b2 = params['ee_b2'].reshape(1, 128)
    p1pe = params['p1_W'][:16][_PE_PERM]
    p1ff = params['p1_W'][16:]
    p1b = params['p1_b'].reshape(1, 128)
    p2pe = params['p2_W'][:16][_PE_PERM]
    p2ff = params['p2_W'][16:]
    p2b = params['p2_b'].reshape(1, 128)
    p3pe = params['p3_W'][:16][_PE_PERM]
    p3ff = params['p3_W'][16:144]
    p3mm = params['p3_W'][144:]
    p3b = params['p3_b'].reshape(1, 128)
    ign_r = params['ign_W'][:128]
    ign_c = params['ign_W'][128:]
    ignb = params['ign_b'].reshape(1, 128)
    cw1 = params['cm_W1']
    cb1 = params['cm_b1'].reshape(1, 128)
    cw2 = params['cm_W2']
    nw1 = params['nd_W1']
    nb1 = params['nd_b1'].reshape(1, 128)
    nw2 = params['nd_W2']
    nb2 = params['nd_b2'].reshape(1, 128)
    one3 = jnp.zeros((1, 16), _F32).at[0, 3].set(1.0)
    angs = jnp.asarray(_ANG_SCALE).reshape(1, 8)

    cd = _sc_gather2(x16, x16, rows, cols, 'sub', 40)
    prod = _sc_gather2(cd, cd, nbr, nbc, 'mul', 80)
    hr, hc = _sc_gather2(h, h, rows, cols, 'none', 40)
    efn = _tc_efn(hr, hc, w1r, w1c, b1, w2, b2)
    ff = _sc_gather2(efn, efn, nbr, nbc, 'mul', 80)
    sincos, m1, m2 = _tc_m1m2(prod, ff, angs, p1pe, p1ff, p1b,
                              p2pe, p2ff, p2b)
    s1, s2 = _sc_scatter_pair(m1, nbr, m2, nbc, E)
    mm = _sc_gather2(s1, s2, nbr, nbc, 'mul', 80)
    nb_x2 = _tc_nbx2(sincos, ff, mm, p3pe, p3ff, p3mm, p3b)
    sr, sc = _sc_scatter_pair(nb_x2, nbr, nb_x2, nbc, E)
    t1, trans16 = _tc_t1(sr, sc, cd, ign_r, ign_c, ignb, cw1, cb1, cw2, one3)
    t0, aggc = _sc_scatter_node(t1, trans16, rows, N)
    h_out, x16_out = _tc_final(x16, aggc, t0, h, nw1, nb1, nw2, nb2)
    return (h_out, x16_out[:, :3], edge_attr)


# speedup vs baseline: 2.2351x; 1.0828x over previous
import functools

import jax
import jax.numpy as jnp
import numpy as np
from jax import lax
from jax.experimental import pallas as pl
from jax.experimental.pallas import tpu as pltpu
from jax.experimental.pallas import tpu_sc as plsc

N = 10000
E = 160000
E2 = 320000

BE = 640
BN = 1000

_F32 = jnp.float32

_DIV = np.exp(np.log(10000.0) * (2.0 * np.arange(8, dtype=np.float32) / 16.0))
_ANG_SCALE = (8.0 / _DIV).astype(np.float32)
_PE_PERM = np.concatenate([np.arange(0, 16, 2), np.arange(1, 16, 2)])


def _silu(v):
    return v * jax.nn.sigmoid(v)


def _row_spec(b, w):
    return pl.BlockSpec((b, w), lambda i: (i, 0))


def _full_spec(shape):
    nd = len(shape)
    return pl.BlockSpec(shape, lambda i: (0,) * nd)


def _efn_body(hr, hc, w1r, w1c, b1, w2, b2, out):
    t = (jnp.dot(hr[...], w1r[...], preferred_element_type=_F32)
         + jnp.dot(hc[...], w1c[...], preferred_element_type=_F32) + b1[...])
    t = _silu(t)
    out[...] = jnp.dot(t, w2[...], preferred_element_type=_F32) + b2[...]


def _tc_efn(hr, hc, w1r, w1c, b1, w2, b2):
    grid = (E // BE,)
    return pl.pallas_call(
        _efn_body,
        grid=grid,
        in_specs=[_row_spec(BE, 128), _row_spec(BE, 128),
                  _full_spec((128, 128)), _full_spec((128, 128)),
                  _full_spec((1, 128)), _full_spec((128, 128)),
                  _full_spec((1, 128))],
        out_specs=_row_spec(BE, 128),
        out_shape=jax.ShapeDtypeStruct((E, 128), _F32),
    )(hr, hc, w1r, w1c, b1, w2, b2)


def _m1m2_body(prod, ff, angs, p1pe, p1ff, p1b, p2pe, p2ff, p2b,
               sincos_out, m1_out, m2_out):
    vtv = jnp.sum(prod[...], axis=1, keepdims=True)
    ang = vtv * angs[...]
    sincos = jnp.concatenate([jnp.sin(ang), jnp.cos(ang)], axis=1)
    sincos_out[...] = sincos
    ffv = ff[...]
    m1 = (jnp.dot(sincos, p1pe[...], preferred_element_type=_F32)
          + jnp.dot(ffv, p1ff[...], preferred_element_type=_F32) + p1b[...])
    m1_out[...] = _silu(m1)
    m2 = (jnp.dot(sincos, p2pe[...], preferred_element_type=_F32)
          + jnp.dot(ffv, p2ff[...], preferred_element_type=_F32) + p2b[...])
    m2_out[...] = _silu(m2)


def _tc_m1m2(prod, ff, angs, p1pe, p1ff, p1b, p2pe, p2ff, p2b):
    grid = (E2 // BE,)
    return pl.pallas_call(
        _m1m2_body,
        grid=grid,
        in_specs=[_row_spec(BE, 16), _row_spec(BE, 128), _full_spec((1, 8)),
                  _full_spec((16, 128)), _full_spec((128, 128)),
                  _full_spec((1, 128)),
                  _full_spec((16, 128)), _full_spec((128, 128)),
                  _full_spec((1, 128))],
        out_specs=[_row_spec(BE, 16), _row_spec(BE, 128), _row_spec(BE, 128)],
        out_shape=[jax.ShapeDtypeStruct((E2, 16), _F32),
                   jax.ShapeDtypeStruct((E2, 128), _F32),
                   jax.ShapeDtypeStruct((E2, 128), _F32)],
    )(prod, ff, angs, p1pe, p1ff, p1b, p2pe, p2ff, p2b)


def _nbx2_body(sincos, ff, mm, p3pe, p3ff, p3mm, p3b, out):
    out[...] = (jnp.dot(sincos[...], p3pe[...], preferred_element_type=_F32)
                + jnp.dot(ff[...], p3ff[...], preferred_element_type=_F32)
                + jnp.dot(mm[...], p3mm[...], preferred_element_type=_F32)
                + p3b[...])


def _tc_nbx2(sincos, ff, mm, p3pe, p3ff, p3mm, p3b):
    grid = (E2 // BE,)
    return pl.pallas_call(
        _nbx2_body,
        grid=grid,
        in_specs=[_row_spec(BE, 16), _row_spec(BE, 128), _row_spec(BE, 128),
                  _full_spec((16, 128)), _full_spec((128, 128)),
                  _full_spec((128, 128)), _full_spec((1, 128))],
        out_specs=_row_spec(BE, 128),
        out_shape=jax.ShapeDtypeStruct((E2, 128), _F32),
    )(sincos, ff, mm, p3pe, p3ff, p3mm, p3b)


def _t1_body(sr, sc, cd, ign_r, ign_c, ignb, cw1, cb1, cw2, one3,
             t1_out, trans_out):
    t1 = (jnp.dot(sr[...], ign_r[...], preferred_element_type=_F32)
          + jnp.dot(sc[...], ign_c[...], preferred_element_type=_F32)
          + ignb[...])
    t1_out[...] = t1
    u = _silu(jnp.dot(t1, cw1[...], preferred_element_type=_F32) + cb1[...])
    w = jnp.dot(u, cw2[...], preferred_element_type=_F32)
    trans_out[...] = cd[...] * w + one3[...]


def _tc_t1(sr, sc, cd, ign_r, ign_c, ignb, cw1, cb1, cw2, one3):
    grid = (E // BE,)
    return pl.pallas_call(
        _t1_body,
        grid=grid,
        in_specs=[_row_spec(BE, 128), _row_spec(BE, 128), _row_spec(BE, 16),
                  _full_spec((128, 128)), _full_spec((128, 128)),
                  _full_spec((1, 128)), _full_spec((128, 128)),
                  _full_spec((1, 128)), _full_spec((128, 1)),
                  _full_spec((1, 16))],
        out_specs=[_row_spec(BE, 128), _row_spec(BE, 16)],
        out_shape=[jax.ShapeDtypeStruct((E, 128), _F32),
                   jax.ShapeDtypeStruct((E, 16), _F32)],
    )(sr, sc, cd, ign_r, ign_c, ignb, cw1, cb1, cw2, one3)


def _final_body(x16, aggc, t0, h, nw1, nb1, nw2, nb2, h_out, x_out):
    cnt = jnp.maximum(aggc[...][:, 3:4], 1.0)
    x_out[...] = x16[...] + aggc[...] * (1.0 / cnt)
    u = _silu(jnp.dot(t0[...], nw1[...], preferred_element_type=_F32)
              + nb1[...])
    h_out[...] = h[...] + jnp.dot(u, nw2[...],
                                  preferred_element_type=_F32) + nb2[...]


def _tc_final(x16, aggc, t0, h, nw1, nb1, nw2, nb2):
    grid = (N // BN,)
    return pl.pallas_call(
        _final_body,
        grid=grid,
        in_specs=[_row_spec(BN, 16), _row_spec(BN, 16), _row_spec(BN, 128),
                  _row_spec(BN, 128),
                  _full_spec((128, 128)), _full_spec((1, 128)),
                  _full_spec((128, 128)), _full_spec((1, 128))],
        out_specs=[_row_spec(BN, 128), _row_spec(BN, 16)],
        out_shape=[jax.ShapeDtypeStruct((N, 128), _F32),
                   jax.ShapeDtypeStruct((N, 16), _F32)],
    )(x16, aggc, t0, h, nw1, nb1, nw2, nb2)


_NC, _NS = 2, 16
_NW = _NC * _NS


def _sc_gather2(t1, t2, idx1, idx2, op, cb):
    B = idx1.shape[0]
    D = t1.shape[1]
    cnt = B // _NW
    nch = cnt // cb
    assert cnt % cb == 0 and cb % 8 == 0 and cb <= 128, (B, cnt, cb)
    mesh = plsc.VectorSubcoreMesh(core_axis_name="c", subcore_axis_name="s")
    nout = 2 if op == 'none' else 1
    if nout == 2:
        outs = (jax.ShapeDtypeStruct((B, D), _F32),
                jax.ShapeDtypeStruct((B, D), _F32))
    else:
        outs = jax.ShapeDtypeStruct((B, D), _F32)

    def body(t1_h, t2_h, i1_h, i2_h, *rest):
        if nout == 2:
            o1_h, o2_h = rest[:2]
            iv1, iv2, b1, b2, sm1, sm2 = rest[2:]
        else:
            o1_h = rest[0]
            o2_h = None
            iv1, iv2, b1, b2, sm1, sm2 = rest[1:]
        wid = lax.axis_index("s") * _NC + lax.axis_index("c")
        base0 = wid * cnt

        def step(k, carry):
            base = base0 + k * cb
            pltpu.sync_copy(i1_h.at[pl.ds(base, cb)], iv1)
            pltpu.sync_copy(i2_h.at[pl.ds(base, cb)], iv2)
            c1 = pltpu.async_copy(t1_h.at[iv1], b1, sm1)
            c2 = pltpu.async_copy(t2_h.at[iv2], b2, sm2)
            c1.wait()
            c2.wait()
            if op != 'none':
                def ew(i, c2_):
                    for g in range(D // 16):
                        sl = (i, pl.ds(g * 16, 16))
                        if op == 'mul':
                            b1[sl] = b1[sl] * b2[sl]
                        else:
                            b1[sl] = b1[sl] - b2[sl]
                    return c2_
                lax.fori_loop(0, cb, ew, 0)
            pltpu.sync_copy(b1, o1_h.at[pl.ds(base, cb)])
            if nout == 2:
                pltpu.sync_copy(b2, o2_h.at[pl.ds(base, cb)])
            return carry

        lax.fori_loop(0, nch, step, 0)

    f = pl.kernel(
        body, out_type=outs, mesh=mesh,
        compiler_params=pltpu.CompilerParams(use_tc_tiling_on_sc=False),
        scratch_types=[
            pltpu.VMEM((cb,), jnp.int32), pltpu.VMEM((cb,), jnp.int32),
            pltpu.VMEM((cb, D), _F32), pltpu.VMEM((cb, D), _F32),
            pltpu.SemaphoreType.DMA, pltpu.SemaphoreType.DMA])
    return f(t1, t2, idx1, idx2)


def _sc_scatter_pair(src1, idx1, src2, idx2, T):
    B = idx1.shape[0]
    IB = 125
    KB = 16
    NB = B // IB
    nbs = NB // _NS
    nout_ch = nbs // KB
    assert B % IB == 0 and NB % _NS == 0 and nbs % KB == 0
    zr = T // _NS
    zc = 500
    nzc = zr // zc
    assert zr % zc == 0
    mesh = plsc.VectorSubcoreMesh(core_axis_name="c", subcore_axis_name="s")
    outs = (jax.ShapeDtypeStruct((T, 128), _F32),
            jax.ShapeDtypeStruct((T, 128), _F32))
    i1_2d = idx1.reshape(NB, IB)
    i2_2d = idx2.reshape(NB, IB)

    def body(s1_h, i1_h, s2_h, i2_h, z_h, o1_h, o2_h,
             table, ivb0, svb0, ivb1, svb1, zb, sma, smb, smc):
        cid = lax.axis_index("c")
        sid = lax.axis_index("s")
        pltpu.sync_copy(z_h, zb)
        for srci in range(2):
            s_h = (s1_h, s2_h)[srci]
            i_h = (i1_h, i2_h)[srci]
            o_h = (o1_h, o2_h)[srci]

            @pl.when(cid == srci)
            def _():
                def stage(o, col, ivb_, svb_, sem_):
                    blk = sid * nbs + o * KB
                    pltpu.async_copy(i_h.at[pl.ds(blk, KB)], ivb_, sem_)
                    pltpu.async_copy(
                        s_h.at[pl.ds(blk * IB, KB * IB), pl.ds(col, 8)],
                        svb_, sem_)

                def wait_stage(ivb_, svb_, sem_):
                    pltpu.make_async_copy(
                        i_h.at[pl.ds(0, KB)], ivb_, sem_).wait()
                    pltpu.make_async_copy(
                        s_h.at[pl.ds(0, KB * IB), pl.ds(0, 8)],
                        svb_, sem_).wait()

                def scat(ivb_, svb_):
                    cps = [pltpu.async_copy(
                               svb_.at[pl.ds(b * IB, IB)],
                               table.at[ivb_.at[b]], smc, add=True)
                           for b in range(KB)]
                    for cp in cps:
                        cp.wait()

                def cpass(cc, carry):
                    col = cc * 8
                    zcps = [pltpu.async_copy(
                                zb, table.at[pl.ds(sid * zr + j * zc, zc)],
                                sma)
                            for j in range(nzc)]
                    for cp in zcps:
                        cp.wait()
                    plsc.subcore_barrier()
                    stage(0, col, ivb0, svb0, sma)

                    def obody(p, c):
                        o0 = 2 * p
                        stage(o0 + 1, col, ivb1, svb1, smb)
                        wait_stage(ivb0, svb0, sma)
                        scat(ivb0, svb0)

                        @pl.when(o0 + 2 < nout_ch)
                        def _():
                            stage(o0 + 2, col, ivb0, svb0, sma)
                        wait_stage(ivb1, svb1, smb)
                        scat(ivb1, svb1)
                        return c
                    lax.fori_loop(0, nout_ch // 2, obody, 0)
                    plsc.subcore_barrier()
                    pltpu.sync_copy(
                        table.at[pl.ds(sid * zr, zr)],
                        o_h.at[pl.ds(sid * zr, zr), pl.ds(col, 8)])
                    return carry

                lax.fori_loop(0, 16, cpass, 0)
                plsc.subcore_barrier()

    f = pl.kernel(
        body, out_type=outs, mesh=mesh,
        compiler_params=pltpu.CompilerParams(use_tc_tiling_on_sc=False),
        scratch_types=[
            pltpu.VMEM_SHARED((T, 8), _F32),
            pltpu.VMEM((KB, IB), jnp.int32),
            pltpu.VMEM((KB * IB, 8), _F32),
            pltpu.VMEM((KB, IB), jnp.int32),
            pltpu.VMEM((KB * IB, 8), _F32),
            pltpu.VMEM((zc, 8), _F32),
            pltpu.SemaphoreType.DMA, pltpu.SemaphoreType.DMA,
            pltpu.SemaphoreType.DMA])
    z = jnp.zeros((zc, 8), _F32)
    return f(src1, i1_2d, src2, i2_2d, z)


def _sc_scatter_node(vals128, vals16, idx, T):
    B = idx.shape[0]
    per = B // _NS
    cbs = 80
    nch = per // cbs
    zr = T // _NS
    assert per % cbs == 0 and T % _NS == 0
    mesh = plsc.VectorSubcoreMesh(core_axis_name="c", subcore_axis_name="s")
    outs = (jax.ShapeDtypeStruct((T, 128), _F32),
            jax.ShapeDtypeStruct((T, 16), _F32))

    def body(v128_h, v16_h, i_h, z_h, o128_h, o16_h,
             tbl128, tbl16, iv, sv128, sv16):
        cid = lax.axis_index("c")
        sid = lax.axis_index("s")

        @pl.when(cid == 0)
        def _():
            for j in range(zr // 125):
                pltpu.sync_copy(
                    z_h, tbl128.at[pl.ds(sid * zr + j * 125, 125)])
            plsc.subcore_barrier()

            def sstep(k, c):
                base = sid * per + k * cbs
                pltpu.sync_copy(i_h.at[pl.ds(base, cbs)], iv)
                pltpu.sync_copy(v128_h.at[pl.ds(base, cbs)], sv128)
                pltpu.sync_copy(sv128, tbl128.at[iv], add=True)
                return c
            lax.fori_loop(0, nch, sstep, 0)
            plsc.subcore_barrier()
            pltpu.sync_copy(tbl128.at[pl.ds(sid * zr, zr)],
                            o128_h.at[pl.ds(sid * zr, zr)])

        @pl.when(cid == 1)
        def _():
            for j in range(zr // 125):
                pltpu.sync_copy(
                    z_h.at[:, pl.ds(0, 16)],
                    tbl16.at[pl.ds(sid * zr + j * 125, 125)])
            plsc.subcore_barrier()

            def sstep(k, c):
                base = sid * per + k * cbs
                pltpu.sync_copy(i_h.at[pl.ds(base, cbs)], iv)
                pltpu.sync_copy(v16_h.at[pl.ds(base, cbs)], sv16)
                pltpu.sync_copy(sv16, tbl16.at[iv], add=True)
                return c
            lax.fori_loop(0, nch, sstep, 0)
            plsc.subcore_barrier()
            pltpu.sync_copy(tbl16.at[pl.ds(sid * zr, zr)],
                            o16_h.at[pl.ds(sid * zr, zr)])

    f = pl.kernel(
        body, out_type=outs, mesh=mesh,
        compiler_params=pltpu.CompilerParams(use_tc_tiling_on_sc=False),
        scratch_types=[
            pltpu.VMEM_SHARED((T, 128), _F32),
            pltpu.VMEM_SHARED((T, 16), _F32),
            pltpu.VMEM((cbs,), jnp.int32),
            pltpu.VMEM((cbs, 128), _F32),
            pltpu.VMEM((cbs, 16), _F32)])
    z = jnp.zeros((125, 128), _F32)
    return f(vals128, vals16, idx, z)


def _gather_rows(table, idx):
    return jnp.take(table, idx, axis=0)


def _segsum(vals, idx, num):
    return jax.ops.segment_sum(vals, idx, num_segments=num)


def kernel(h, x, edges, nb_edge, edge_attr, nb_num_nodes, params):
    del nb_num_nodes
    rows, cols = edges[0], edges[1]
    nbr, nbc = nb_edge[0], nb_edge[1]

    x16 = jnp.pad(x, ((0, 0), (0, 13)))

    w1r = params['ee_W1'][:128]
    w1c = params['ee_W1'][128:]
    b1 = params['ee_b1'].reshape(1, 128)
    w2 = params['ee_W2']
    b2 = params['ee_b2'].reshape(1, 128)
    p1pe = params['p1_W'][:16][_PE_PERM]
    p1ff = params['p1_W'][16:]
    p1b = params['p1_b'].reshape(1, 128)
    p2pe = params['p2_W'][:16][_PE_PERM]
    p2ff = params['p2_W'][16:]
    p2b = params['p2_b'].reshape(1, 128)
    p3pe = params['p3_W'][:16][_PE_PERM]
    p3ff = params['p3_W'][16:144]
    p3mm = params['p3_W'][144:]
    p3b = params['p3_b'].reshape(1, 128)
    ign_r = params['ign_W'][:128]
    ign_c = params['ign_W'][128:]
    ignb = params['ign_b'].reshape(1, 128)
    cw1 = params['cm_W1']
    cb1 = params['cm_b1'].reshape(1, 128)
    cw2 = params['cm_W2']
    nw1 = params['nd_W1']
    nb1 = params['nd_b1'].reshape(1, 128)
    nw2 = params['nd_W2']
    nb2 = params['nd_b2'].reshape(1, 128)
    one3 = jnp.zeros((1, 16), _F32).at[0, 3].set(1.0)
    angs = jnp.asarray(_ANG_SCALE).reshape(1, 8)

    cd = _sc_gather2(x16, x16, rows, cols, 'sub', 40)
    prod = _sc_gather2(cd, cd, nbr, nbc, 'mul', 80)
    hr, hc = _sc_gather2(h, h, rows, cols, 'none', 40)
    efn = _tc_efn(hr, hc, w1r, w1c, b1, w2, b2)
    ff = _sc_gather2(efn, efn, nbr, nbc, 'mul', 80)
    sincos, m1, m2 = _tc_m1m2(prod, ff, angs, p1pe, p1ff, p1b,
                              p2pe, p2ff, p2b)
    s1, s2 = _sc_scatter_pair(m1, nbr, m2, nbc, E)
    mm = _sc_gather2(s1, s2, nbr, nbc, 'mul', 80)
    nb_x2 = _tc_nbx2(sincos, ff, mm, p3pe, p3ff, p3mm, p3b)
    sr, sc = _sc_scatter_pair(nb_x2, nbr, nb_x2, nbc, E)
    t1, trans16 = _tc_t1(sr, sc, cd, ign_r, ign_c, ignb, cw1, cb1, cw2, one3)
    t0, aggc = _sc_scatter_node(t1, trans16, rows, N)
    h_out, x16_out = _tc_final(x16, aggc, t0, h, nw1, nb1, nw2, nb2)
    return (h_out, x16_out[:, :3], edge_attr)


# speedup vs baseline: 2.5898x; 1.1587x over previous
import functools

import jax
import jax.numpy as jnp
import numpy as np
from jax import lax
from jax.experimental import pallas as pl
from jax.experimental.pallas import tpu as pltpu
from jax.experimental.pallas import tpu_sc as plsc

N = 10000
E = 160000
E2 = 320000

BE = 640
BN = 1000

_F32 = jnp.float32

_DIV = np.exp(np.log(10000.0) * (2.0 * np.arange(8, dtype=np.float32) / 16.0))
_ANG_SCALE = (8.0 / _DIV).astype(np.float32)
_PE_PERM = np.concatenate([np.arange(0, 16, 2), np.arange(1, 16, 2)])


def _silu(v):
    return v * jax.nn.sigmoid(v)


def _row_spec(b, w):
    return pl.BlockSpec((b, w), lambda i: (i, 0))


def _full_spec(shape):
    nd = len(shape)
    return pl.BlockSpec(shape, lambda i: (0,) * nd)


def _efn_body(hr, hc, w1r, w1c, b1, w2, b2, out):
    t = (jnp.dot(hr[...], w1r[...], preferred_element_type=_F32)
         + jnp.dot(hc[...], w1c[...], preferred_element_type=_F32) + b1[...])
    t = _silu(t)
    out[...] = jnp.dot(t, w2[...], preferred_element_type=_F32) + b2[...]


def _tc_efn(hr, hc, w1r, w1c, b1, w2, b2):
    grid = (E // BE,)
    return pl.pallas_call(
        _efn_body,
        grid=grid,
        in_specs=[_row_spec(BE, 128), _row_spec(BE, 128),
                  _full_spec((128, 128)), _full_spec((128, 128)),
                  _full_spec((1, 128)), _full_spec((128, 128)),
                  _full_spec((1, 128))],
        out_specs=_row_spec(BE, 128),
        out_shape=jax.ShapeDtypeStruct((E, 128), _F32),
    )(hr, hc, w1r, w1c, b1, w2, b2)


def _m1m2_body(prod, ff, angs, p1pe, p1ff, p1b, p2pe, p2ff, p2b,
               sincos_out, m1_out, m2_out):
    vtv = jnp.sum(prod[...], axis=1, keepdims=True)
    ang = vtv * angs[...]
    sincos = jnp.concatenate([jnp.sin(ang), jnp.cos(ang)], axis=1)
    sincos_out[...] = sincos
    ffv = ff[...]
    m1 = (jnp.dot(sincos, p1pe[...], preferred_element_type=_F32)
          + jnp.dot(ffv, p1ff[...], preferred_element_type=_F32) + p1b[...])
    m1_out[...] = _silu(m1)
    m2 = (jnp.dot(sincos, p2pe[...], preferred_element_type=_F32)
          + jnp.dot(ffv, p2ff[...], preferred_element_type=_F32) + p2b[...])
    m2_out[...] = _silu(m2)


def _tc_m1m2(prod, ff, angs, p1pe, p1ff, p1b, p2pe, p2ff, p2b):
    grid = (E2 // BE,)
    return pl.pallas_call(
        _m1m2_body,
        grid=grid,
        in_specs=[_row_spec(BE, 16), _row_spec(BE, 128), _full_spec((1, 8)),
                  _full_spec((16, 128)), _full_spec((128, 128)),
                  _full_spec((1, 128)),
                  _full_spec((16, 128)), _full_spec((128, 128)),
                  _full_spec((1, 128))],
        out_specs=[_row_spec(BE, 16), _row_spec(BE, 128), _row_spec(BE, 128)],
        out_shape=[jax.ShapeDtypeStruct((E2, 16), _F32),
                   jax.ShapeDtypeStruct((E2, 128), _F32),
                   jax.ShapeDtypeStruct((E2, 128), _F32)],
    )(prod, ff, angs, p1pe, p1ff, p1b, p2pe, p2ff, p2b)


def _nbx2_body(sincos, ff, mm, p3pe, p3ff, p3mm, p3b, out):
    out[...] = (jnp.dot(sincos[...], p3pe[...], preferred_element_type=_F32)
                + jnp.dot(ff[...], p3ff[...], preferred_element_type=_F32)
                + jnp.dot(mm[...], p3mm[...], preferred_element_type=_F32)
                + p3b[...])


def _tc_nbx2(sincos, ff, mm, p3pe, p3ff, p3mm, p3b):
    grid = (E2 // BE,)
    return pl.pallas_call(
        _nbx2_body,
        grid=grid,
        in_specs=[_row_spec(BE, 16), _row_spec(BE, 128), _row_spec(BE, 128),
                  _full_spec((16, 128)), _full_spec((128, 128)),
                  _full_spec((128, 128)), _full_spec((1, 128))],
        out_specs=_row_spec(BE, 128),
        out_shape=jax.ShapeDtypeStruct((E2, 128), _F32),
    )(sincos, ff, mm, p3pe, p3ff, p3mm, p3b)


def _t1_body(sr, sc, cd, ign_r, ign_c, ignb, cw1, cb1, cw2, one3,
             t1_out, trans_out):
    t1 = (jnp.dot(sr[...], ign_r[...], preferred_element_type=_F32)
          + jnp.dot(sc[...], ign_c[...], preferred_element_type=_F32)
          + ignb[...])
    t1_out[...] = t1
    u = _silu(jnp.dot(t1, cw1[...], preferred_element_type=_F32) + cb1[...])
    w = jnp.dot(u, cw2[...], preferred_element_type=_F32)
    trans_out[...] = cd[...] * w + one3[...]


def _tc_t1(sr, sc, cd, ign_r, ign_c, ignb, cw1, cb1, cw2, one3):
    grid = (E // BE,)
    return pl.pallas_call(
        _t1_body,
        grid=grid,
        in_specs=[_row_spec(BE, 128), _row_spec(BE, 128), _row_spec(BE, 16),
                  _full_spec((128, 128)), _full_spec((128, 128)),
                  _full_spec((1, 128)), _full_spec((128, 128)),
                  _full_spec((1, 128)), _full_spec((128, 1)),
                  _full_spec((1, 16))],
        out_specs=[_row_spec(BE, 128), _row_spec(BE, 16)],
        out_shape=[jax.ShapeDtypeStruct((E, 128), _F32),
                   jax.ShapeDtypeStruct((E, 16), _F32)],
    )(sr, sc, cd, ign_r, ign_c, ignb, cw1, cb1, cw2, one3)


def _final_body(x16, aggc, t0, h, nw1, nb1, nw2, nb2, h_out, x_out):
    cnt = jnp.maximum(aggc[...][:, 3:4], 1.0)
    x_out[...] = x16[...] + aggc[...] * (1.0 / cnt)
    u = _silu(jnp.dot(t0[...], nw1[...], preferred_element_type=_F32)
              + nb1[...])
    h_out[...] = h[...] + jnp.dot(u, nw2[...],
                                  preferred_element_type=_F32) + nb2[...]


def _tc_final(x16, aggc, t0, h, nw1, nb1, nw2, nb2):
    grid = (N // BN,)
    return pl.pallas_call(
        _final_body,
        grid=grid,
        in_specs=[_row_spec(BN, 16), _row_spec(BN, 16), _row_spec(BN, 128),
                  _row_spec(BN, 128),
                  _full_spec((128, 128)), _full_spec((1, 128)),
                  _full_spec((128, 128)), _full_spec((1, 128))],
        out_specs=[_row_spec(BN, 128), _row_spec(BN, 16)],
        out_shape=[jax.ShapeDtypeStruct((N, 128), _F32),
                   jax.ShapeDtypeStruct((N, 16), _F32)],
    )(x16, aggc, t0, h, nw1, nb1, nw2, nb2)


_NC, _NS = 2, 16
_NW = _NC * _NS


def _sc_gather2(t1, t2, idx1, idx2, op):
    B = idx1.shape[0]
    D = t1.shape[1]
    IB = 125
    NBg = B // IB
    nbw = NBg // _NW
    assert B % IB == 0 and NBg % _NW == 0 and nbw % 2 == 0
    mesh = plsc.VectorSubcoreMesh(core_axis_name="c", subcore_axis_name="s")
    nout = 2 if op == 'none' else 1
    if nout == 2:
        outs = (jax.ShapeDtypeStruct((B, D), _F32),
                jax.ShapeDtypeStruct((B, D), _F32))
    else:
        outs = jax.ShapeDtypeStruct((B, D), _F32)
    i1_2d = idx1.reshape(NBg, IB)
    i2_2d = idx2.reshape(NBg, IB)

    def body(t1_h, t2_h, i1_h, i2_h, *rest):
        if nout == 2:
            o1_h, o2_h = rest[:2]
            rest = rest[2:]
        else:
            o1_h = rest[0]
            o2_h = None
            rest = rest[1:]
        iv1, iv2, b1a, b2a, b1b, b2b, sga, sgb = rest
        wid = lax.axis_index("s") * _NC + lax.axis_index("c")
        blk0 = wid * nbw
        pltpu.sync_copy(i1_h.at[pl.ds(blk0, nbw)], iv1)
        pltpu.sync_copy(i2_h.at[pl.ds(blk0, nbw)], iv2)

        def fire(o, b1_, b2_, sem_):
            pltpu.async_copy(t1_h.at[iv1.at[o]], b1_, sem_)
            pltpu.async_copy(t2_h.at[iv2.at[o]], b2_, sem_)

        def wait_g(b1_, b2_, sem_):
            pltpu.make_async_copy(t1_h.at[iv1.at[0]], b1_, sem_).wait()
            pltpu.make_async_copy(t2_h.at[iv2.at[0]], b2_, sem_).wait()

        def finish(o, b1_, b2_):
            if op != 'none':
                def ew(i, c):
                    for g in range(D // 16):
                        sl = (i, pl.ds(g * 16, 16))
                        if op == 'mul':
                            b1_[sl] = b1_[sl] * b2_[sl]
                        else:
                            b1_[sl] = b1_[sl] - b2_[sl]
                    return c
                lax.fori_loop(0, IB, ew, 0)
            pltpu.sync_copy(b1_, o1_h.at[pl.ds((blk0 + o) * IB, IB)])
            if nout == 2:
                pltpu.sync_copy(b2_, o2_h.at[pl.ds((blk0 + o) * IB, IB)])

        fire(0, b1a, b2a, sga)

        def obody(p, c):
            o0 = 2 * p
            fire(o0 + 1, b1b, b2b, sgb)
            wait_g(b1a, b2a, sga)
            finish(o0, b1a, b2a)

            @pl.when(o0 + 2 < nbw)
            def _():
                fire(o0 + 2, b1a, b2a, sga)
            wait_g(b1b, b2b, sgb)
            finish(o0 + 1, b1b, b2b)
            return c
        lax.fori_loop(0, nbw // 2, obody, 0)

    f = pl.kernel(
        body, out_type=outs, mesh=mesh,
        compiler_params=pltpu.CompilerParams(use_tc_tiling_on_sc=False),
        scratch_types=[
            pltpu.VMEM((nbw, IB), jnp.int32), pltpu.VMEM((nbw, IB), jnp.int32),
            pltpu.VMEM((IB, D), _F32), pltpu.VMEM((IB, D), _F32),
            pltpu.VMEM((IB, D), _F32), pltpu.VMEM((IB, D), _F32),
            pltpu.SemaphoreType.DMA, pltpu.SemaphoreType.DMA])
    return f(t1, t2, i1_2d, i2_2d)


def _sc_scatter_pair(src1, idx1, src2, idx2, T):
    B = idx1.shape[0]
    IB = 125
    KB = 16
    NB = B // IB
    nbs = NB // _NS
    nout_ch = nbs // KB
    assert B % IB == 0 and NB % _NS == 0 and nbs % KB == 0
    zr = T // _NS
    zc = 500
    nzc = zr // zc
    assert zr % zc == 0
    mesh = plsc.VectorSubcoreMesh(core_axis_name="c", subcore_axis_name="s")
    outs = (jax.ShapeDtypeStruct((T, 128), _F32),
            jax.ShapeDtypeStruct((T, 128), _F32))
    i1_2d = idx1.reshape(NB, IB)
    i2_2d = idx2.reshape(NB, IB)

    def body(s1_h, i1_h, s2_h, i2_h, z_h, o1_h, o2_h,
             table, ivb0, svb0, ivb1, svb1, zb, sma, smb, smc):
        cid = lax.axis_index("c")
        sid = lax.axis_index("s")
        pltpu.sync_copy(z_h, zb)
        for srci in range(2):
            s_h = (s1_h, s2_h)[srci]
            i_h = (i1_h, i2_h)[srci]
            o_h = (o1_h, o2_h)[srci]

            @pl.when(cid == srci)
            def _():
                def stage(o, col, ivb_, svb_, sem_):
                    blk = sid * nbs + o * KB
                    pltpu.async_copy(i_h.at[pl.ds(blk, KB)], ivb_, sem_)
                    pltpu.async_copy(
                        s_h.at[pl.ds(blk * IB, KB * IB), pl.ds(col, 8)],
                        svb_, sem_)

                def wait_stage(ivb_, svb_, sem_):
                    pltpu.make_async_copy(
                        i_h.at[pl.ds(0, KB)], ivb_, sem_).wait()
                    pltpu.make_async_copy(
                        s_h.at[pl.ds(0, KB * IB), pl.ds(0, 8)],
                        svb_, sem_).wait()

                def scat(ivb_, svb_):
                    cps = [pltpu.async_copy(
                               svb_.at[pl.ds(b * IB, IB)],
                               table.at[ivb_.at[b]], smc, add=True)
                           for b in range(KB)]
                    for cp in cps:
                        cp.wait()

                def cpass(cc, carry):
                    col = cc * 8
                    zcps = [pltpu.async_copy(
                                zb, table.at[pl.ds(sid * zr + j * zc, zc)],
                                sma)
                            for j in range(nzc)]
                    for cp in zcps:
                        cp.wait()
                    plsc.subcore_barrier()
                    stage(0, col, ivb0, svb0, sma)

                    def obody(p, c):
                        o0 = 2 * p
                        stage(o0 + 1, col, ivb1, svb1, smb)
                        wait_stage(ivb0, svb0, sma)
                        scat(ivb0, svb0)

                        @pl.when(o0 + 2 < nout_ch)
                        def _():
                            stage(o0 + 2, col, ivb0, svb0, sma)
                        wait_stage(ivb1, svb1, smb)
                        scat(ivb1, svb1)
                        return c
                    lax.fori_loop(0, nout_ch // 2, obody, 0)
                    plsc.subcore_barrier()
                    pltpu.sync_copy(
                        table.at[pl.ds(sid * zr, zr)],
                        o_h.at[pl.ds(sid * zr, zr), pl.ds(col, 8)])
                    return carry

                lax.fori_loop(0, 16, cpass, 0)
                plsc.subcore_barrier()

    f = pl.kernel(
        body, out_type=outs, mesh=mesh,
        compiler_params=pltpu.CompilerParams(use_tc_tiling_on_sc=False),
        scratch_types=[
            pltpu.VMEM_SHARED((T, 8), _F32),
            pltpu.VMEM((KB, IB), jnp.int32),
            pltpu.VMEM((KB * IB, 8), _F32),
            pltpu.VMEM((KB, IB), jnp.int32),
            pltpu.VMEM((KB * IB, 8), _F32),
            pltpu.VMEM((zc, 8), _F32),
            pltpu.SemaphoreType.DMA, pltpu.SemaphoreType.DMA,
            pltpu.SemaphoreType.DMA])
    z = jnp.zeros((zc, 8), _F32)
    return f(src1, i1_2d, src2, i2_2d, z)


def _sc_scatter_node(vals128, vals16, idx, T):
    B = idx.shape[0]
    per = B // _NS
    cbs = 80
    nch = per // cbs
    zr = T // _NS
    assert per % cbs == 0 and T % _NS == 0
    mesh = plsc.VectorSubcoreMesh(core_axis_name="c", subcore_axis_name="s")
    outs = (jax.ShapeDtypeStruct((T, 128), _F32),
            jax.ShapeDtypeStruct((T, 16), _F32))

    def body(v128_h, v16_h, i_h, z_h, o128_h, o16_h,
             tbl128, tbl16, iv, sv128, sv16):
        cid = lax.axis_index("c")
        sid = lax.axis_index("s")

        @pl.when(cid == 0)
        def _():
            for j in range(zr // 125):
                pltpu.sync_copy(
                    z_h, tbl128.at[pl.ds(sid * zr + j * 125, 125)])
            plsc.subcore_barrier()

            def sstep(k, c):
                base = sid * per + k * cbs
                pltpu.sync_copy(i_h.at[pl.ds(base, cbs)], iv)
                pltpu.sync_copy(v128_h.at[pl.ds(base, cbs)], sv128)
                pltpu.sync_copy(sv128, tbl128.at[iv], add=True)
                return c
            lax.fori_loop(0, nch, sstep, 0)
            plsc.subcore_barrier()
            pltpu.sync_copy(tbl128.at[pl.ds(sid * zr, zr)],
                            o128_h.at[pl.ds(sid * zr, zr)])

        @pl.when(cid == 1)
        def _():
            for j in range(zr // 125):
                pltpu.sync_copy(
                    z_h.at[:, pl.ds(0, 16)],
                    tbl16.at[pl.ds(sid * zr + j * 125, 125)])
            plsc.subcore_barrier()

            def sstep(k, c):
                base = sid * per + k * cbs
                pltpu.sync_copy(i_h.at[pl.ds(base, cbs)], iv)
                pltpu.sync_copy(v16_h.at[pl.ds(base, cbs)], sv16)
                pltpu.sync_copy(sv16, tbl16.at[iv], add=True)
                return c
            lax.fori_loop(0, nch, sstep, 0)
            plsc.subcore_barrier()
            pltpu.sync_copy(tbl16.at[pl.ds(sid * zr, zr)],
                            o16_h.at[pl.ds(sid * zr, zr)])

    f = pl.kernel(
        body, out_type=outs, mesh=mesh,
        compiler_params=pltpu.CompilerParams(use_tc_tiling_on_sc=False),
        scratch_types=[
            pltpu.VMEM_SHARED((T, 128), _F32),
            pltpu.VMEM_SHARED((T, 16), _F32),
            pltpu.VMEM((cbs,), jnp.int32),
            pltpu.VMEM((cbs, 128), _F32),
            pltpu.VMEM((cbs, 16), _F32)])
    z = jnp.zeros((125, 128), _F32)
    return f(vals128, vals16, idx, z)


def _gather_rows(table, idx):
    return jnp.take(table, idx, axis=0)


def _segsum(vals, idx, num):
    return jax.ops.segment_sum(vals, idx, num_segments=num)


def kernel(h, x, edges, nb_edge, edge_attr, nb_num_nodes, params):
    del nb_num_nodes
    rows, cols = edges[0], edges[1]
    nbr, nbc = nb_edge[0], nb_edge[1]

    x16 = jnp.pad(x, ((0, 0), (0, 13)))

    w1r = params['ee_W1'][:128]
    w1c = params['ee_W1'][128:]
    b1 = params['ee_b1'].reshape(1, 128)
    w2 = params['ee_W2']
    b2 = params['ee_b2'].reshape(1, 128)
    p1pe = params['p1_W'][:16][_PE_PERM]
    p1ff = params['p1_W'][16:]
    p1b = params['p1_b'].reshape(1, 128)
    p2pe = params['p2_W'][:16][_PE_PERM]
    p2ff = params['p2_W'][16:]
    p2b = params['p2_b'].reshape(1, 128)
    p3pe = params['p3_W'][:16][_PE_PERM]
    p3ff = params['p3_W'][16:144]
    p3mm = params['p3_W'][144:]
    p3b = params['p3_b'].reshape(1, 128)
    ign_r = params['ign_W'][:128]
    ign_c = params['ign_W'][128:]
    ignb = params['ign_b'].reshape(1, 128)
    cw1 = params['cm_W1']
    cb1 = params['cm_b1'].reshape(1, 128)
    cw2 = params['cm_W2']
    nw1 = params['nd_W1']
    nb1 = params['nd_b1'].reshape(1, 128)
    nw2 = params['nd_W2']
    nb2 = params['nd_b2'].reshape(1, 128)
    one3 = jnp.zeros((1, 16), _F32).at[0, 3].set(1.0)
    angs = jnp.asarray(_ANG_SCALE).reshape(1, 8)

    cd = _sc_gather2(x16, x16, rows, cols, 'sub')
    prod = _sc_gather2(cd, cd, nbr, nbc, 'mul')
    hr, hc = _sc_gather2(h, h, rows, cols, 'none')
    efn = _tc_efn(hr, hc, w1r, w1c, b1, w2, b2)
    ff = _sc_gather2(efn, efn, nbr, nbc, 'mul')
    sincos, m1, m2 = _tc_m1m2(prod, ff, angs, p1pe, p1ff, p1b,
                              p2pe, p2ff, p2b)
    s1, s2 = _sc_scatter_pair(m1, nbr, m2, nbc, E)
    mm = _sc_gather2(s1, s2, nbr, nbc, 'mul')
    nb_x2 = _tc_nbx2(sincos, ff, mm, p3pe, p3ff, p3mm, p3b)
    sr, sc = _sc_scatter_pair(nb_x2, nbr, nb_x2, nbc, E)
    t1, trans16 = _tc_t1(sr, sc, cd, ign_r, ign_c, ignb, cw1, cb1, cw2, one3)
    t0, aggc = _sc_scatter_node(t1, trans16, rows, N)
    h_out, x16_out = _tc_final(x16, aggc, t0, h, nw1, nb1, nw2, nb2)
    return (h_out, x16_out[:, :3], edge_attr)
